# Initial kernel scaffold; baseline (speedup 1.0000x reference)
import jax
import jax.numpy as jnp
from jax.experimental import pallas as pl


def kernel(input, W, W_new):
    raise NotImplementedError("write your pallas kernel here")


# speedup vs baseline: 5.6785x; 5.6785x over previous
import functools

import jax
import jax.numpy as jnp
from jax import lax
from jax.experimental import pallas as pl
from jax.experimental.pallas import tpu as pltpu
from jax.experimental.pallas import tpu_sc as plsc

_VOCAB = 100000
_LANES = 16
_SUB = 128


def _emb_kernel(num_rows, dim, chunk, n_workers, num_cores):
  per_w = num_rows // n_workers
  n_chunks = per_w // chunk
  n_sub = chunk // _SUB
  groups = chunk // _LANES

  mesh = plsc.VectorSubcoreMesh(core_axis_name="c", subcore_axis_name="s")

  @functools.partial(
      pl.kernel,
      mesh=mesh,
      compiler_params=pltpu.CompilerParams(
          needs_layout_passes=False, use_tc_tiling_on_sc=False
      ),
      out_type=jax.ShapeDtypeStruct((num_rows, dim), jnp.float32),
      scratch_types=[
          pltpu.VMEM((chunk,), jnp.int32),
          pltpu.VMEM((chunk,), jnp.int32),
          pltpu.VMEM((chunk, dim), jnp.float32),
          pltpu.VMEM((_SUB, dim), jnp.float32),
          pltpu.SemaphoreType.DMA,
      ],
  )
  def body(ids_hbm, w_hbm, wn_hbm, out_hbm, idx_v, idxa_v, rows_v, wn_v, sem):
    wid = lax.axis_index("s") * num_cores + lax.axis_index("c")
    base = wid * per_w

    pltpu.sync_copy(wn_hbm, wn_v.at[pl.ds(0, wn_hbm.shape[0])])

    def chunk_body(g, carry):
      off = base + g * chunk
      pltpu.sync_copy(ids_hbm.at[pl.ds(off, chunk)], idx_v)

      def clamp_body(j, c):
        v = idx_v[pl.ds(j * _LANES, _LANES)]
        idxa_v[pl.ds(j * _LANES, _LANES)] = jnp.where(v >= _VOCAB, 0, v)
        return c

      lax.fori_loop(0, groups, clamp_body, 0, unroll=False)

      handles = []
      for k in range(n_sub):
        handles.append(
            pltpu.async_copy(
                w_hbm.at[idxa_v.at[pl.ds(k * _SUB, _SUB)]],
                rows_v.at[pl.ds(k * _SUB, _SUB)],
                sem,
            )
        )
      for h in handles:
        h.wait()

      lanes = lax.iota(jnp.int32, _LANES)

      def fix_group(j, c):
        v = idx_v[pl.ds(j * _LANES, _LANES)]
        hit = v >= _VOCAB

        @pl.when(jnp.any(hit))
        def _():
          jj = jnp.where(hit, v - _VOCAB, 0)
          r_vec = j * _LANES + lanes

          def fix_col(cc, c2):
            c_vec = jnp.zeros((_LANES,), jnp.int32) + cc
            vals = plsc.load_gather(wn_v, [jj, c_vec], mask=hit)
            plsc.store_scatter(rows_v, [r_vec, c_vec], vals, mask=hit)
            return c2

          lax.fori_loop(0, dim, fix_col, 0, unroll=False)

        return c

      lax.fori_loop(0, groups, fix_group, 0, unroll=False)

      pltpu.sync_copy(rows_v, out_hbm.at[pl.ds(off, chunk)])
      return carry

    lax.fori_loop(0, n_chunks, chunk_body, 0, unroll=False)

  return body


def kernel(input, W, W_new):
  b, l = input.shape
  vocab, dim = W.shape
  num_rows = b * l
  info = plsc.get_sparse_core_info()
  n_workers = info.num_cores * info.num_subcores
  ids = input.reshape(num_rows).astype(jnp.int32)
  fn = _emb_kernel(num_rows, dim, 1024, n_workers, info.num_cores)
  out = fn(ids, W, W_new)
  return out.reshape(b, l, dim)
